# baseline (speedup 1.0000x reference)
import functools

import jax
import jax.numpy as jnp
from jax import lax
from jax.experimental import pallas as pl
from jax.experimental.pallas import tpu as pltpu
from jax.experimental.pallas import tpu_sc as plsc

B = 4096
S = 200
D = 64
H = 512
C = 10
CPAD = 128

NC = 2
NS = 16
NW = NC * NS
BPW = B // NW
S2 = S // 2
NJ = D // 16

_mesh = plsc.VectorSubcoreMesh(core_axis_name="c", subcore_axis_name="s")


@functools.partial(
    pl.kernel,
    mesh=_mesh,
    compiler_params=pltpu.CompilerParams(use_tc_tiling_on_sc=False),
    out_type=jax.ShapeDtypeStruct((B, D), jnp.float32),
    scratch_types=[
        pltpu.VMEM((BPW, 2, S2), jnp.int32),
        pltpu.VMEM((2, 2, S2, D), jnp.float32),
        pltpu.VMEM((BPW, D), jnp.float32),
        pltpu.SemaphoreType.DMA,
        pltpu.SemaphoreType.DMA,
    ],
)
def _pool(x_hbm, emb_hbm, dummy_hbm, out_hbm, idx_v, rows_v, out_v, sem0, sem1):
    wid = lax.axis_index("s") * NC + lax.axis_index("c")
    row0 = wid * BPW
    pltpu.sync_copy(x_hbm.at[pl.ds(row0, BPW)], idx_v)

    def gather(r, buf, sem):
        pltpu.async_copy(emb_hbm.at[idx_v.at[r, 0]], rows_v.at[buf, 0], sem)
        pltpu.async_copy(emb_hbm.at[idx_v.at[r, 1]], rows_v.at[buf, 1], sem)

    def wait_gather(buf, sem):
        for half in range(2):
            pltpu.make_async_copy(dummy_hbm, rows_v.at[buf, half], sem).wait()

    def accumulate(r, buf):
        def acc_body(i, accs):
            out = []
            for j in range(NJ):
                a = accs[j]
                a = a + rows_v[buf, 0, i, pl.ds(j * 16, 16)]
                a = a + rows_v[buf, 1, i, pl.ds(j * 16, 16)]
                out.append(a)
            return tuple(out)

        zeros = tuple(jnp.zeros((16,), jnp.float32) for _ in range(NJ))
        accs = lax.fori_loop(0, S2, acc_body, zeros, unroll=2)
        for j in range(NJ):
            out_v[r, pl.ds(j * 16, 16)] = accs[j] * (1.0 / S)

    gather(0, 0, sem0)

    def pair_body(p, carry):
        r = 2 * p
        gather(r + 1, 1, sem1)
        wait_gather(0, sem0)
        accumulate(r, 0)

        @pl.when(p < BPW // 2 - 1)
        def _():
            gather(r + 2, 0, sem0)

        wait_gather(1, sem1)
        accumulate(r + 1, 1)
        return carry

    lax.fori_loop(0, BPW // 2, pair_body, 0)
    pltpu.sync_copy(out_v, out_hbm.at[pl.ds(row0, BPW)])


def _mlp_body(p_ref, w1_ref, b1_ref, w2_ref, b2_ref, o_ref):
    h = jnp.dot(p_ref[:], w1_ref[:], preferred_element_type=jnp.float32)
    h = jnp.maximum(h + b1_ref[:], 0.0)
    o_ref[:] = jnp.dot(h, w2_ref[:], preferred_element_type=jnp.float32) + b2_ref[:]


BT = 1024


def _mlp(pooled, W1, b1, W2, b2):
    W2p = jnp.zeros((H, CPAD), jnp.float32).at[:, :C].set(W2)
    b2p = jnp.zeros((1, CPAD), jnp.float32).at[:, :C].set(b2)
    out = pl.pallas_call(
        _mlp_body,
        grid=(B // BT,),
        in_specs=[
            pl.BlockSpec((BT, D), lambda i: (i, 0)),
            pl.BlockSpec((D, H), lambda i: (0, 0)),
            pl.BlockSpec((1, H), lambda i: (0, 0)),
            pl.BlockSpec((H, CPAD), lambda i: (0, 0)),
            pl.BlockSpec((1, CPAD), lambda i: (0, 0)),
        ],
        out_specs=pl.BlockSpec((BT, CPAD), lambda i: (i, 0)),
        out_shape=jax.ShapeDtypeStruct((B, CPAD), jnp.float32),
    )(pooled, W1, b1.reshape(1, H), W2p, b2p)
    return out[:, :C]


def kernel(x, emb, W1, b1, W2, b2):
    x3 = x.astype(jnp.int32).reshape(B, 2, S2)
    dummy = jnp.zeros((S2, D), jnp.float32)
    pooled = _pool(x3, emb, dummy)
    return _mlp(pooled, W1, b1, W2, b2)
